# baseline (speedup 1.0000x reference)
import dataclasses
import functools

import jax
import jax.numpy as jnp
from jax import lax
from jax.experimental import pallas as pl
from jax.experimental.pallas import tpu as pltpu
from jax.experimental.pallas import tpu_sc as plsc

N_NODES = 10000
N_EDGES = 160000
C = 256
HALF = 128
GAMMA = 0.1
EPS = 0.1

NC = 2
NS = 16
CHUNK = 40
E_PER_SUB = N_EDGES // NS
N_CHUNKS = E_PER_SUB // CHUNK
E_PER_WORKER = N_EDGES // (NC * NS)
N_PAD = 10240
STRIPE = N_PAD // NS

_mesh = plsc.VectorSubcoreMesh(core_axis_name="c", subcore_axis_name="s")


D_ROWS0 = 128
D_ROWS1 = N_CHUNKS - D_ROWS0

_cp = pltpu.CompilerParams()
if "needs_layout_passes" in pltpu.CompilerParams.__dataclass_fields__:
    _cp = dataclasses.replace(_cp, needs_layout_passes=False)


@functools.partial(
    pl.kernel,
    mesh=_mesh,
    compiler_params=_cp,
    out_type=jax.ShapeDtypeStruct((NC, N_PAD), jnp.float32),
    scratch_types=[
        pltpu.VMEM((D_ROWS0, CHUNK), jnp.int32),
        pltpu.VMEM((N_PAD,), jnp.float32),
        pltpu.VMEM((STRIPE,), jnp.float32),
        pltpu.VMEM((STRIPE,), jnp.float32),
        pltpu.VMEM_SHARED((NS, N_PAD), jnp.float32),
    ],
)
def _deg_kernel(col_hbm, out_hbm, col_v, hist, tmp, accs, stage_sh):
    cid = lax.axis_index("c")
    sid = lax.axis_index("s")

    @pl.when(cid == 0)
    def _():
        pltpu.sync_copy(col_hbm.at[sid, pl.ds(0, D_ROWS0)], col_v)

    @pl.when(cid == 1)
    def _():
        pltpu.sync_copy(col_hbm.at[sid, pl.ds(D_ROWS0, D_ROWS1)],
                        col_v.at[pl.ds(0, D_ROWS1)])

    @pl.loop(0, N_PAD // 16)
    def _(i):
        hist[pl.ds(i * 16, 16)] = jnp.zeros((16,), jnp.float32)

    one16 = jnp.ones((16,), jnp.float32)
    tail_mask = lax.iota(jnp.int32, 16) >= 8

    nrows = jnp.where(cid == 0, D_ROWS0, D_ROWS1)

    @pl.loop(0, D_ROWS0)
    def _(r):
        row_mask = jnp.where(r < nrows, tail_mask, jnp.zeros((16,), jnp.bool_))
        full_mask = jnp.where(r < nrows, jnp.ones((16,), jnp.bool_),
                              jnp.zeros((16,), jnp.bool_))
        plsc.addupdate_scatter(hist, [col_v[r, pl.ds(0, 16)]], one16,
                               mask=full_mask)
        plsc.addupdate_scatter(hist, [col_v[r, pl.ds(16, 16)]], one16,
                               mask=full_mask)
        plsc.addupdate_scatter(hist, [col_v[r, pl.ds(24, 16)]], one16,
                               mask=row_mask)

    pltpu.sync_copy(hist, stage_sh.at[sid])
    plsc.subcore_barrier()

    @pl.loop(0, STRIPE // 16)
    def _(t):
        accs[pl.ds(t * 16, 16)] = jnp.zeros((16,), jnp.float32)

    @pl.loop(0, NS)
    def _(k):
        pltpu.sync_copy(stage_sh.at[k, pl.ds(sid * STRIPE, STRIPE)], tmp)

        @pl.loop(0, STRIPE // 16)
        def _(t):
            sl = pl.ds(t * 16, 16)
            accs[sl] = accs[sl] + tmp[sl]

    pltpu.sync_copy(accs, out_hbm.at[cid, pl.ds(sid * STRIPE, STRIPE)])


@functools.partial(
    pl.kernel,
    mesh=_mesh,
    out_type=jax.ShapeDtypeStruct((NC, N_PAD, HALF), jnp.float32),
    scratch_types=[
        pltpu.VMEM((N_CHUNKS, CHUNK), jnp.int32),
        pltpu.VMEM((2, CHUNK), jnp.int32),
        pltpu.VMEM((CHUNK, HALF), jnp.float32),
        pltpu.VMEM((CHUNK, HALF), jnp.float32),
        pltpu.VMEM_SHARED((N_PAD, HALF), jnp.float32),
        pltpu.SemaphoreType.DMA,
        pltpu.SemaphoreType.DMA,
        pltpu.SemaphoreType.DMA,
        pltpu.SemaphoreType.DMA,
        pltpu.SemaphoreType.DMA,
        pltpu.SemaphoreType.DMA,
    ],
)
def _agg_kernel(hs_hbm, row_hbm, col_hbm, out_hbm, row_v, col_v, buf_a, buf_b,
                acc_sh, sem_ga, sem_gb, sem_ca, sem_cb, sem_sa, sem_sb):
    cid = lax.axis_index("c")
    sid = lax.axis_index("s")
    hs_c = hs_hbm.at[cid]
    pltpu.sync_copy(row_hbm.at[sid], row_v)

    @pl.when(sid < NS - 1)
    def _():
        pltpu.sync_copy(hs_c.at[pl.ds(sid * STRIPE, STRIPE)],
                        acc_sh.at[pl.ds(sid * STRIPE, STRIPE)])

    @pl.when(sid == NS - 1)
    def _():
        pltpu.sync_copy(
            hs_c.at[pl.ds((NS - 1) * STRIPE, N_NODES - (NS - 1) * STRIPE)],
            acc_sh.at[pl.ds((NS - 1) * STRIPE, N_NODES - (NS - 1) * STRIPE)])

    plsc.subcore_barrier()

    pltpu.async_copy(col_hbm.at[sid, 0], col_v.at[0], sem_ca)
    pltpu.async_copy(col_hbm.at[sid, 1], col_v.at[1], sem_cb)
    pltpu.async_copy(hs_c.at[row_v.at[0]], buf_a, sem_ga)
    pltpu.async_copy(hs_c.at[row_v.at[1]], buf_b, sem_gb)

    @pl.loop(0, N_CHUNKS, step=2)
    def _(j):
        ja = jnp.minimum(j + 2, N_CHUNKS - 1)
        jb = jnp.minimum(j + 3, N_CHUNKS - 1)
        pltpu.make_async_copy(hs_c.at[row_v.at[0]], buf_a, sem_ga).wait()
        pltpu.make_async_copy(col_hbm.at[sid, 0], col_v.at[0], sem_ca).wait()
        pltpu.async_copy(buf_a, acc_sh.at[col_v.at[0]], sem_sa, add=True)
        pltpu.make_async_copy(hs_c.at[row_v.at[0]], buf_b, sem_gb).wait()
        pltpu.make_async_copy(col_hbm.at[sid, 0], col_v.at[1], sem_cb).wait()
        pltpu.async_copy(buf_b, acc_sh.at[col_v.at[1]], sem_sb, add=True)
        pltpu.make_async_copy(buf_a, acc_sh.at[col_v.at[0]], sem_sa).wait()
        pltpu.async_copy(col_hbm.at[sid, ja], col_v.at[0], sem_ca)
        pltpu.async_copy(hs_c.at[row_v.at[ja]], buf_a, sem_ga)
        pltpu.make_async_copy(buf_b, acc_sh.at[col_v.at[1]], sem_sb).wait()
        pltpu.async_copy(col_hbm.at[sid, jb], col_v.at[1], sem_cb)
        pltpu.async_copy(hs_c.at[row_v.at[jb]], buf_b, sem_gb)

    pltpu.make_async_copy(hs_c.at[row_v.at[0]], buf_a, sem_ga).wait()
    pltpu.make_async_copy(hs_c.at[row_v.at[0]], buf_b, sem_gb).wait()
    pltpu.make_async_copy(col_hbm.at[sid, 0], col_v.at[0], sem_ca).wait()
    pltpu.make_async_copy(col_hbm.at[sid, 0], col_v.at[1], sem_cb).wait()

    plsc.subcore_barrier()
    pltpu.sync_copy(
        acc_sh.at[pl.ds(sid * STRIPE, STRIPE)],
        out_hbm.at[cid, pl.ds(sid * STRIPE, STRIPE)],
    )


_BLK = 1000


def _mmhs_body(x_ref, wphi_ref, deg_ref, hs_ref):
    d = deg_ref[:, 0:1] + deg_ref[:, 1:2] + 1.0
    dinv = lax.rsqrt(d)
    h = jnp.dot(x_ref[...], wphi_ref[...], preferred_element_type=jnp.float32)
    hs_ref[0] = dinv * h[:, :HALF]
    hs_ref[1] = dinv * h[:, HALF:]


def _mmhs_call(x, wphi, deg2):
    return pl.pallas_call(
        _mmhs_body,
        grid=(N_NODES // _BLK,),
        in_specs=[
            pl.BlockSpec((_BLK, C), lambda i: (i, 0)),
            pl.BlockSpec((C, C), lambda i: (0, 0)),
            pl.BlockSpec((_BLK, NC), lambda i: (i, 0)),
        ],
        out_specs=pl.BlockSpec((NC, _BLK, HALF), lambda i: (0, i, 0)),
        out_shape=jax.ShapeDtypeStruct((NC, N_NODES, HALF), jnp.float32),
    )(x, wphi, deg2)


def _fin_body(x_ref, at_ref, b_ref, acc_ref, deg_ref, o_ref):
    xb = x_ref[...]
    d = deg_ref[:, 0:1] + deg_ref[:, 1:2] + 1.0
    dinv = lax.rsqrt(d)
    z = jnp.dot(xb, at_ref[...], preferred_element_type=jnp.float32) + b_ref[...]
    accf = jnp.concatenate([acc_ref[0], acc_ref[1]], axis=-1)
    o_ref[...] = xb + EPS * jnp.tanh(z + dinv * accf)


def _fin_call(x, w, b2, acc, deg2):
    return pl.pallas_call(
        _fin_body,
        grid=(N_NODES // _BLK,),
        in_specs=[
            pl.BlockSpec((_BLK, C), lambda i: (i, 0)),
            pl.BlockSpec((C, C), lambda i: (0, 0)),
            pl.BlockSpec((1, C), lambda i: (0, 0)),
            pl.BlockSpec((NC, _BLK, HALF), lambda i: (0, i, 0)),
            pl.BlockSpec((_BLK, NC), lambda i: (i, 0)),
        ],
        out_specs=pl.BlockSpec((_BLK, C), lambda i: (i, 0)),
        out_shape=jax.ShapeDtypeStruct((N_NODES, C), jnp.float32),
    )(x, w, b2, acc, deg2)


def kernel(x, edge_index, W, b, W_phi):
    ei = edge_index.astype(jnp.int32)
    row = ei[0]
    col = ei[1]
    col3 = col.reshape(NS, N_CHUNKS, CHUNK)
    row3 = row.reshape(NS, N_CHUNKS, CHUNK)
    a_t = W.T - W - GAMMA * jnp.eye(C, dtype=W.dtype)

    deg2 = _deg_kernel(col3).T
    hs2 = _mmhs_call(x, W_phi, deg2)
    acc = _agg_kernel(hs2, row3, col3)
    return _fin_call(x, a_t, b.reshape(1, C), acc, deg2)


# speedup vs baseline: 1.2181x; 1.2181x over previous
import dataclasses
import functools

import jax
import jax.numpy as jnp
from jax import lax
from jax.experimental import pallas as pl
from jax.experimental.pallas import tpu as pltpu
from jax.experimental.pallas import tpu_sc as plsc

N_NODES = 10000
N_EDGES = 160000
C = 256
HALF = 128
GAMMA = 0.1
EPS = 0.1

NC = 2
NS = 16
CHUNK = 125
E_PER_SUB = N_EDGES // NS
N_CHUNKS = E_PER_SUB // CHUNK
E_PER_WORKER = N_EDGES // (NC * NS)
N_PAD = 10240
STRIPE = N_PAD // NS

_mesh = plsc.VectorSubcoreMesh(core_axis_name="c", subcore_axis_name="s")


D_ROWS = N_CHUNKS // NC

_cp = pltpu.CompilerParams()
if "needs_layout_passes" in pltpu.CompilerParams.__dataclass_fields__:
    _cp = dataclasses.replace(_cp, needs_layout_passes=False)


@functools.partial(
    pl.kernel,
    mesh=_mesh,
    compiler_params=_cp,
    out_type=jax.ShapeDtypeStruct((NC, N_PAD), jnp.float32),
    scratch_types=[
        pltpu.VMEM((D_ROWS, CHUNK), jnp.int32),
        pltpu.VMEM((N_PAD,), jnp.float32),
        pltpu.VMEM((STRIPE,), jnp.float32),
        pltpu.VMEM((STRIPE,), jnp.float32),
        pltpu.VMEM_SHARED((NS, N_PAD), jnp.float32),
    ],
)
def _deg_kernel(col_hbm, out_hbm, col_v, hist, tmp, accs, stage_sh):
    cid = lax.axis_index("c")
    sid = lax.axis_index("s")
    pltpu.sync_copy(col_hbm.at[sid, pl.ds(cid * D_ROWS, D_ROWS)], col_v)

    @pl.loop(0, N_PAD // 16)
    def _(i):
        hist[pl.ds(i * 16, 16)] = jnp.zeros((16,), jnp.float32)

    one16 = jnp.ones((16,), jnp.float32)
    tail_mask = lax.iota(jnp.int32, 16) < CHUNK - 112
    tail_cols = jnp.minimum(112 + lax.iota(jnp.int32, 16), CHUNK - 1)

    @pl.loop(0, D_ROWS)
    def _(r):
        @pl.loop(0, 7)
        def _(k):
            plsc.addupdate_scatter(hist, [col_v[r, pl.ds(k * 16, 16)]], one16)

        rows16 = jnp.full((16,), r, jnp.int32)
        tvals = plsc.load_gather(col_v, [rows16, tail_cols])
        plsc.addupdate_scatter(hist, [tvals], one16, mask=tail_mask)

    pltpu.sync_copy(hist, stage_sh.at[sid])
    plsc.subcore_barrier()

    @pl.loop(0, STRIPE // 16)
    def _(t):
        accs[pl.ds(t * 16, 16)] = jnp.zeros((16,), jnp.float32)

    @pl.loop(0, NS)
    def _(k):
        pltpu.sync_copy(stage_sh.at[k, pl.ds(sid * STRIPE, STRIPE)], tmp)

        @pl.loop(0, STRIPE // 16)
        def _(t):
            sl = pl.ds(t * 16, 16)
            accs[sl] = accs[sl] + tmp[sl]

    pltpu.sync_copy(accs, out_hbm.at[cid, pl.ds(sid * STRIPE, STRIPE)])


@functools.partial(
    pl.kernel,
    mesh=_mesh,
    out_type=jax.ShapeDtypeStruct((NC, N_PAD, HALF), jnp.float32),
    scratch_types=[
        pltpu.VMEM((N_CHUNKS, CHUNK), jnp.int32),
        pltpu.VMEM((2, CHUNK), jnp.int32),
        pltpu.VMEM((CHUNK, HALF), jnp.float32),
        pltpu.VMEM((CHUNK, HALF), jnp.float32),
        pltpu.VMEM_SHARED((N_PAD, HALF), jnp.float32),
        pltpu.SemaphoreType.DMA,
        pltpu.SemaphoreType.DMA,
        pltpu.SemaphoreType.DMA,
        pltpu.SemaphoreType.DMA,
        pltpu.SemaphoreType.DMA,
        pltpu.SemaphoreType.DMA,
    ],
)
def _agg_kernel(hs_hbm, row_hbm, col_hbm, out_hbm, row_v, col_v, buf_a, buf_b,
                acc_sh, sem_ga, sem_gb, sem_ca, sem_cb, sem_sa, sem_sb):
    cid = lax.axis_index("c")
    sid = lax.axis_index("s")
    hs_c = hs_hbm.at[cid]
    pltpu.sync_copy(row_hbm.at[sid], row_v)

    @pl.when(sid < NS - 1)
    def _():
        pltpu.sync_copy(hs_c.at[pl.ds(sid * STRIPE, STRIPE)],
                        acc_sh.at[pl.ds(sid * STRIPE, STRIPE)])

    @pl.when(sid == NS - 1)
    def _():
        pltpu.sync_copy(
            hs_c.at[pl.ds((NS - 1) * STRIPE, N_NODES - (NS - 1) * STRIPE)],
            acc_sh.at[pl.ds((NS - 1) * STRIPE, N_NODES - (NS - 1) * STRIPE)])

    plsc.subcore_barrier()

    pltpu.async_copy(col_hbm.at[sid, 0], col_v.at[0], sem_ca)
    pltpu.async_copy(col_hbm.at[sid, 1], col_v.at[1], sem_cb)
    pltpu.async_copy(hs_c.at[row_v.at[0]], buf_a, sem_ga)
    pltpu.async_copy(hs_c.at[row_v.at[1]], buf_b, sem_gb)

    @pl.loop(0, N_CHUNKS, step=2)
    def _(j):
        ja = jnp.minimum(j + 2, N_CHUNKS - 1)
        jb = jnp.minimum(j + 3, N_CHUNKS - 1)
        pltpu.make_async_copy(hs_c.at[row_v.at[0]], buf_a, sem_ga).wait()
        pltpu.make_async_copy(col_hbm.at[sid, 0], col_v.at[0], sem_ca).wait()
        pltpu.async_copy(buf_a, acc_sh.at[col_v.at[0]], sem_sa, add=True)
        pltpu.make_async_copy(hs_c.at[row_v.at[0]], buf_b, sem_gb).wait()
        pltpu.make_async_copy(col_hbm.at[sid, 0], col_v.at[1], sem_cb).wait()
        pltpu.async_copy(buf_b, acc_sh.at[col_v.at[1]], sem_sb, add=True)
        pltpu.make_async_copy(buf_a, acc_sh.at[col_v.at[0]], sem_sa).wait()
        pltpu.async_copy(col_hbm.at[sid, ja], col_v.at[0], sem_ca)
        pltpu.async_copy(hs_c.at[row_v.at[ja]], buf_a, sem_ga)
        pltpu.make_async_copy(buf_b, acc_sh.at[col_v.at[1]], sem_sb).wait()
        pltpu.async_copy(col_hbm.at[sid, jb], col_v.at[1], sem_cb)
        pltpu.async_copy(hs_c.at[row_v.at[jb]], buf_b, sem_gb)

    pltpu.make_async_copy(hs_c.at[row_v.at[0]], buf_a, sem_ga).wait()
    pltpu.make_async_copy(hs_c.at[row_v.at[0]], buf_b, sem_gb).wait()
    pltpu.make_async_copy(col_hbm.at[sid, 0], col_v.at[0], sem_ca).wait()
    pltpu.make_async_copy(col_hbm.at[sid, 0], col_v.at[1], sem_cb).wait()

    plsc.subcore_barrier()
    pltpu.sync_copy(
        acc_sh.at[pl.ds(sid * STRIPE, STRIPE)],
        out_hbm.at[cid, pl.ds(sid * STRIPE, STRIPE)],
    )


_BLK = 2000


def _mm_body(x_ref, wphi_ref, h_ref):
    h_ref[...] = jnp.dot(x_ref[...], wphi_ref[...],
                         preferred_element_type=jnp.float32)


def _mm_call(x, wphi):
    return pl.pallas_call(
        _mm_body,
        grid=(N_NODES // _BLK,),
        in_specs=[
            pl.BlockSpec((_BLK, C), lambda i: (i, 0)),
            pl.BlockSpec((C, C), lambda i: (0, 0)),
        ],
        out_specs=pl.BlockSpec((_BLK, C), lambda i: (i, 0)),
        out_shape=jax.ShapeDtypeStruct((N_NODES, C), jnp.float32),
    )(x, wphi)


def _hs_body(h_ref, deg_ref, hs_ref):
    d = deg_ref[:, 0:1] + deg_ref[:, 1:2] + 1.0
    dinv = lax.rsqrt(d)
    h = h_ref[...]
    hs_ref[0] = dinv * h[:, :HALF]
    hs_ref[1] = dinv * h[:, HALF:]


def _hs_call(h, deg2):
    return pl.pallas_call(
        _hs_body,
        grid=(N_NODES // _BLK,),
        in_specs=[
            pl.BlockSpec((_BLK, C), lambda i: (i, 0)),
            pl.BlockSpec((_BLK, NC), lambda i: (i, 0)),
        ],
        out_specs=pl.BlockSpec((NC, _BLK, HALF), lambda i: (0, i, 0)),
        out_shape=jax.ShapeDtypeStruct((NC, N_NODES, HALF), jnp.float32),
    )(h, deg2)


def _fin_body(x_ref, at_ref, b_ref, acc_ref, deg_ref, o_ref):
    xb = x_ref[...]
    d = deg_ref[:, 0:1] + deg_ref[:, 1:2] + 1.0
    dinv = lax.rsqrt(d)
    z = jnp.dot(xb, at_ref[...], preferred_element_type=jnp.float32) + b_ref[...]
    accf = jnp.concatenate([acc_ref[0], acc_ref[1]], axis=-1)
    o_ref[...] = xb + EPS * jnp.tanh(z + dinv * accf)


def _fin_call(x, w, b2, acc, deg2):
    return pl.pallas_call(
        _fin_body,
        grid=(N_NODES // _BLK,),
        in_specs=[
            pl.BlockSpec((_BLK, C), lambda i: (i, 0)),
            pl.BlockSpec((C, C), lambda i: (0, 0)),
            pl.BlockSpec((1, C), lambda i: (0, 0)),
            pl.BlockSpec((NC, _BLK, HALF), lambda i: (0, i, 0)),
            pl.BlockSpec((_BLK, NC), lambda i: (i, 0)),
        ],
        out_specs=pl.BlockSpec((_BLK, C), lambda i: (i, 0)),
        out_shape=jax.ShapeDtypeStruct((N_NODES, C), jnp.float32),
    )(x, w, b2, acc, deg2)


def kernel(x, edge_index, W, b, W_phi):
    ei = edge_index.astype(jnp.int32)
    row = ei[0]
    col = ei[1]
    col3 = col.reshape(NS, N_CHUNKS, CHUNK)
    row3 = row.reshape(NS, N_CHUNKS, CHUNK)
    a_t = W.T - W - GAMMA * jnp.eye(C, dtype=W.dtype)

    deg2 = _deg_kernel(col3).T
    h = _mm_call(x, W_phi)
    hs2 = _hs_call(h, deg2)
    acc = _agg_kernel(hs2, row3, col3)
    return _fin_call(x, a_t, b.reshape(1, C), acc, deg2)


# speedup vs baseline: 1.2369x; 1.0154x over previous
import dataclasses
import functools

import jax
import jax.numpy as jnp
from jax import lax
from jax.experimental import pallas as pl
from jax.experimental.pallas import tpu as pltpu
from jax.experimental.pallas import tpu_sc as plsc

N_NODES = 10000
N_EDGES = 160000
C = 256
HALF = 128
GAMMA = 0.1
EPS = 0.1

NC = 2
NS = 16
CHUNK = 125
E_PER_SUB = N_EDGES // NS
N_CHUNKS = E_PER_SUB // CHUNK
E_PER_WORKER = N_EDGES // (NC * NS)
N_PAD = 10240
STRIPE = N_PAD // NS

_mesh = plsc.VectorSubcoreMesh(core_axis_name="c", subcore_axis_name="s")


D_ROWS = N_CHUNKS // NC

_cp = pltpu.CompilerParams()
if "needs_layout_passes" in pltpu.CompilerParams.__dataclass_fields__:
    _cp = dataclasses.replace(_cp, needs_layout_passes=False)


@functools.partial(
    pl.kernel,
    mesh=_mesh,
    compiler_params=_cp,
    out_type=jax.ShapeDtypeStruct((NC, N_PAD), jnp.float32),
    scratch_types=[
        pltpu.VMEM((D_ROWS, CHUNK), jnp.int32),
        pltpu.VMEM((N_PAD,), jnp.float32),
        pltpu.VMEM((STRIPE,), jnp.float32),
        pltpu.VMEM((STRIPE,), jnp.float32),
        pltpu.VMEM_SHARED((NS, N_PAD), jnp.float32),
    ],
)
def _deg_kernel(e_hbm, out_hbm, col_v, hist, tmp, accs, stage_sh):
    cid = lax.axis_index("c")
    sid = lax.axis_index("s")
    pltpu.sync_copy(e_hbm.at[1, sid, pl.ds(cid * D_ROWS, D_ROWS)], col_v)

    @pl.loop(0, N_PAD // 16)
    def _(i):
        hist[pl.ds(i * 16, 16)] = jnp.zeros((16,), jnp.float32)

    one16 = jnp.ones((16,), jnp.float32)
    tail_mask = lax.iota(jnp.int32, 16) < CHUNK - 112
    tail_cols = jnp.minimum(112 + lax.iota(jnp.int32, 16), CHUNK - 1)

    @pl.loop(0, D_ROWS)
    def _(r):
        @pl.loop(0, 7)
        def _(k):
            plsc.addupdate_scatter(hist, [col_v[r, pl.ds(k * 16, 16)]], one16)

        rows16 = jnp.full((16,), r, jnp.int32)
        tvals = plsc.load_gather(col_v, [rows16, tail_cols])
        plsc.addupdate_scatter(hist, [tvals], one16, mask=tail_mask)

    pltpu.sync_copy(hist, stage_sh.at[sid])
    plsc.subcore_barrier()

    @pl.loop(0, STRIPE // 16)
    def _(t):
        accs[pl.ds(t * 16, 16)] = jnp.zeros((16,), jnp.float32)

    @pl.loop(0, NS)
    def _(k):
        pltpu.sync_copy(stage_sh.at[k, pl.ds(sid * STRIPE, STRIPE)], tmp)

        @pl.loop(0, STRIPE // 16)
        def _(t):
            sl = pl.ds(t * 16, 16)
            accs[sl] = accs[sl] + tmp[sl]

    pltpu.sync_copy(accs, out_hbm.at[cid, pl.ds(sid * STRIPE, STRIPE)])


@functools.partial(
    pl.kernel,
    mesh=_mesh,
    out_type=jax.ShapeDtypeStruct((NC, N_PAD, HALF), jnp.float32),
    scratch_types=[
        pltpu.VMEM((N_CHUNKS, CHUNK), jnp.int32),
        pltpu.VMEM((2, CHUNK), jnp.int32),
        pltpu.VMEM((CHUNK, HALF), jnp.float32),
        pltpu.VMEM((CHUNK, HALF), jnp.float32),
        pltpu.VMEM_SHARED((N_PAD, HALF), jnp.float32),
        pltpu.SemaphoreType.DMA,
        pltpu.SemaphoreType.DMA,
        pltpu.SemaphoreType.DMA,
        pltpu.SemaphoreType.DMA,
        pltpu.SemaphoreType.DMA,
        pltpu.SemaphoreType.DMA,
    ],
)
def _agg_kernel(hs_hbm, e_hbm, out_hbm, row_v, col_v, buf_a, buf_b,
                acc_sh, sem_ga, sem_gb, sem_ca, sem_cb, sem_sa, sem_sb):
    cid = lax.axis_index("c")
    sid = lax.axis_index("s")
    hs_c = hs_hbm.at[cid]
    col_hbm = e_hbm.at[1]
    pltpu.sync_copy(e_hbm.at[0, sid], row_v)

    @pl.when(sid < NS - 1)
    def _():
        pltpu.sync_copy(hs_c.at[pl.ds(sid * STRIPE, STRIPE)],
                        acc_sh.at[pl.ds(sid * STRIPE, STRIPE)])

    @pl.when(sid == NS - 1)
    def _():
        pltpu.sync_copy(
            hs_c.at[pl.ds((NS - 1) * STRIPE, N_NODES - (NS - 1) * STRIPE)],
            acc_sh.at[pl.ds((NS - 1) * STRIPE, N_NODES - (NS - 1) * STRIPE)])

    plsc.subcore_barrier()

    pltpu.async_copy(col_hbm.at[sid, 0], col_v.at[0], sem_ca)
    pltpu.async_copy(col_hbm.at[sid, 1], col_v.at[1], sem_cb)
    pltpu.async_copy(hs_c.at[row_v.at[0]], buf_a, sem_ga)
    pltpu.async_copy(hs_c.at[row_v.at[1]], buf_b, sem_gb)

    @pl.loop(0, N_CHUNKS, step=2)
    def _(j):
        ja = jnp.minimum(j + 2, N_CHUNKS - 1)
        jb = jnp.minimum(j + 3, N_CHUNKS - 1)
        pltpu.make_async_copy(hs_c.at[row_v.at[0]], buf_a, sem_ga).wait()
        pltpu.make_async_copy(col_hbm.at[sid, 0], col_v.at[0], sem_ca).wait()
        pltpu.async_copy(buf_a, acc_sh.at[col_v.at[0]], sem_sa, add=True)
        pltpu.make_async_copy(hs_c.at[row_v.at[0]], buf_b, sem_gb).wait()
        pltpu.make_async_copy(col_hbm.at[sid, 0], col_v.at[1], sem_cb).wait()
        pltpu.async_copy(buf_b, acc_sh.at[col_v.at[1]], sem_sb, add=True)
        pltpu.make_async_copy(buf_a, acc_sh.at[col_v.at[0]], sem_sa).wait()
        pltpu.async_copy(col_hbm.at[sid, ja], col_v.at[0], sem_ca)
        pltpu.async_copy(hs_c.at[row_v.at[ja]], buf_a, sem_ga)
        pltpu.make_async_copy(buf_b, acc_sh.at[col_v.at[1]], sem_sb).wait()
        pltpu.async_copy(col_hbm.at[sid, jb], col_v.at[1], sem_cb)
        pltpu.async_copy(hs_c.at[row_v.at[jb]], buf_b, sem_gb)

    pltpu.make_async_copy(hs_c.at[row_v.at[0]], buf_a, sem_ga).wait()
    pltpu.make_async_copy(hs_c.at[row_v.at[0]], buf_b, sem_gb).wait()
    pltpu.make_async_copy(col_hbm.at[sid, 0], col_v.at[0], sem_ca).wait()
    pltpu.make_async_copy(col_hbm.at[sid, 0], col_v.at[1], sem_cb).wait()

    plsc.subcore_barrier()
    pltpu.sync_copy(
        acc_sh.at[pl.ds(sid * STRIPE, STRIPE)],
        out_hbm.at[cid, pl.ds(sid * STRIPE, STRIPE)],
    )


_BLK = 2000


def _mm_body(x_ref, wphi_ref, h_ref):
    h_ref[...] = jnp.dot(x_ref[...], wphi_ref[...],
                         preferred_element_type=jnp.float32)


def _mm_call(x, wphi):
    return pl.pallas_call(
        _mm_body,
        grid=(N_NODES // _BLK,),
        in_specs=[
            pl.BlockSpec((_BLK, C), lambda i: (i, 0)),
            pl.BlockSpec((C, C), lambda i: (0, 0)),
        ],
        out_specs=pl.BlockSpec((_BLK, C), lambda i: (i, 0)),
        out_shape=jax.ShapeDtypeStruct((N_NODES, C), jnp.float32),
    )(x, wphi)


def _hs_body(h_ref, deg_ref, hs_ref):
    d = deg_ref[:, 0:1] + deg_ref[:, 1:2] + 1.0
    dinv = lax.rsqrt(d)
    h = h_ref[...]
    hs_ref[0] = dinv * h[:, :HALF]
    hs_ref[1] = dinv * h[:, HALF:]


def _hs_call(h, deg2):
    return pl.pallas_call(
        _hs_body,
        grid=(N_NODES // _BLK,),
        in_specs=[
            pl.BlockSpec((_BLK, C), lambda i: (i, 0)),
            pl.BlockSpec((_BLK, NC), lambda i: (i, 0)),
        ],
        out_specs=pl.BlockSpec((NC, _BLK, HALF), lambda i: (0, i, 0)),
        out_shape=jax.ShapeDtypeStruct((NC, N_NODES, HALF), jnp.float32),
    )(h, deg2)


def _fin_body(x_ref, at_ref, b_ref, acc_ref, deg_ref, o_ref):
    xb = x_ref[...]
    d = deg_ref[:, 0:1] + deg_ref[:, 1:2] + 1.0
    dinv = lax.rsqrt(d)
    z = jnp.dot(xb, at_ref[...], preferred_element_type=jnp.float32) + b_ref[...]
    accf = jnp.concatenate([acc_ref[0], acc_ref[1]], axis=-1)
    o_ref[...] = xb + EPS * jnp.tanh(z + dinv * accf)


def _fin_call(x, w, b2, acc, deg2):
    return pl.pallas_call(
        _fin_body,
        grid=(N_NODES // _BLK,),
        in_specs=[
            pl.BlockSpec((_BLK, C), lambda i: (i, 0)),
            pl.BlockSpec((C, C), lambda i: (0, 0)),
            pl.BlockSpec((1, C), lambda i: (0, 0)),
            pl.BlockSpec((NC, _BLK, HALF), lambda i: (0, i, 0)),
            pl.BlockSpec((_BLK, NC), lambda i: (i, 0)),
        ],
        out_specs=pl.BlockSpec((_BLK, C), lambda i: (i, 0)),
        out_shape=jax.ShapeDtypeStruct((N_NODES, C), jnp.float32),
    )(x, w, b2, acc, deg2)


def kernel(x, edge_index, W, b, W_phi):
    e3 = edge_index.astype(jnp.int32).reshape(2, NS, N_CHUNKS, CHUNK)
    a_t = W.T - W - GAMMA * jnp.eye(C, dtype=W.dtype)

    deg2 = _deg_kernel(e3).T
    h = _mm_call(x, W_phi)
    hs2 = _hs_call(h, deg2)
    acc = _agg_kernel(hs2, e3)
    return _fin_call(x, a_t, b.reshape(1, C), acc, deg2)


# speedup vs baseline: 1.2420x; 1.0042x over previous
import dataclasses
import functools

import jax
import jax.numpy as jnp
from jax import lax
from jax.experimental import pallas as pl
from jax.experimental.pallas import tpu as pltpu
from jax.experimental.pallas import tpu_sc as plsc

N_NODES = 10000
N_EDGES = 160000
C = 256
HALF = 128
GAMMA = 0.1
EPS = 0.1

NC = 2
NS = 16
CHUNK = 125
E_PER_SUB = N_EDGES // NS
N_CHUNKS = E_PER_SUB // CHUNK
N_PAD = 10240
STRIPE = N_PAD // NS

_mesh = plsc.VectorSubcoreMesh(core_axis_name="c", subcore_axis_name="s")


D_ROWS = N_CHUNKS // NC

_cp = pltpu.CompilerParams()
if "needs_layout_passes" in pltpu.CompilerParams.__dataclass_fields__:
    _cp = dataclasses.replace(_cp, needs_layout_passes=False)


@functools.partial(
    pl.kernel,
    mesh=_mesh,
    compiler_params=_cp,
    out_type=jax.ShapeDtypeStruct((NC, N_PAD), jnp.float32),
    scratch_types=[
        pltpu.VMEM((D_ROWS, CHUNK), jnp.int32),
        pltpu.VMEM((N_PAD,), jnp.float32),
        pltpu.VMEM((STRIPE,), jnp.float32),
        pltpu.VMEM((STRIPE,), jnp.float32),
        pltpu.VMEM_SHARED((NS, N_PAD), jnp.float32),
    ],
)
def _deg_kernel(e_hbm, out_hbm, col_v, hist, tmp, accs, stage_sh):
    cid = lax.axis_index("c")
    sid = lax.axis_index("s")
    pltpu.sync_copy(e_hbm.at[1, sid, pl.ds(cid * D_ROWS, D_ROWS)], col_v)

    @pl.loop(0, N_PAD // 16)
    def _(i):
        hist[pl.ds(i * 16, 16)] = jnp.zeros((16,), jnp.float32)

    one16 = jnp.ones((16,), jnp.float32)
    tail_mask = lax.iota(jnp.int32, 16) < CHUNK - 112
    tail_cols = jnp.minimum(112 + lax.iota(jnp.int32, 16), CHUNK - 1)

    @pl.loop(0, D_ROWS)
    def _(r):
        @pl.loop(0, 7)
        def _(k):
            plsc.addupdate_scatter(hist, [col_v[r, pl.ds(k * 16, 16)]], one16)

        rows16 = jnp.full((16,), r, jnp.int32)
        tvals = plsc.load_gather(col_v, [rows16, tail_cols])
        plsc.addupdate_scatter(hist, [tvals], one16, mask=tail_mask)

    pltpu.sync_copy(hist, stage_sh.at[sid])
    plsc.subcore_barrier()

    @pl.loop(0, STRIPE // 16)
    def _(t):
        accs[pl.ds(t * 16, 16)] = jnp.zeros((16,), jnp.float32)

    @pl.loop(0, NS)
    def _(k):
        pltpu.sync_copy(stage_sh.at[k, pl.ds(sid * STRIPE, STRIPE)], tmp)

        @pl.loop(0, STRIPE // 16)
        def _(t):
            sl = pl.ds(t * 16, 16)
            accs[sl] = accs[sl] + tmp[sl]

    pltpu.sync_copy(accs, out_hbm.at[cid, pl.ds(sid * STRIPE, STRIPE)])


@functools.partial(
    pl.kernel,
    mesh=_mesh,
    out_type=jax.ShapeDtypeStruct((NC, N_PAD, HALF), jnp.float32),
    scratch_types=[
        pltpu.VMEM((N_CHUNKS, CHUNK), jnp.int32),
        pltpu.VMEM((2, CHUNK), jnp.int32),
        pltpu.VMEM((CHUNK, HALF), jnp.float32),
        pltpu.VMEM((CHUNK, HALF), jnp.float32),
        pltpu.VMEM_SHARED((N_PAD, HALF), jnp.float32),
        pltpu.SemaphoreType.DMA,
        pltpu.SemaphoreType.DMA,
        pltpu.SemaphoreType.DMA,
        pltpu.SemaphoreType.DMA,
        pltpu.SemaphoreType.DMA,
        pltpu.SemaphoreType.DMA,
    ],
)
def _agg_kernel(hs_hbm, e_hbm, out_hbm, row_v, col_v, buf_a, buf_b,
                acc_sh, sem_ga, sem_gb, sem_ca, sem_cb, sem_sa, sem_sb):
    cid = lax.axis_index("c")
    sid = lax.axis_index("s")
    hs_c = hs_hbm.at[cid]
    col_hbm = e_hbm.at[1]
    pltpu.sync_copy(e_hbm.at[0, sid], row_v)

    @pl.when(sid < NS - 1)
    def _():
        pltpu.sync_copy(hs_c.at[pl.ds(sid * STRIPE, STRIPE)],
                        acc_sh.at[pl.ds(sid * STRIPE, STRIPE)])

    @pl.when(sid == NS - 1)
    def _():
        pltpu.sync_copy(
            hs_c.at[pl.ds((NS - 1) * STRIPE, N_NODES - (NS - 1) * STRIPE)],
            acc_sh.at[pl.ds((NS - 1) * STRIPE, N_NODES - (NS - 1) * STRIPE)])

    plsc.subcore_barrier()

    pltpu.async_copy(col_hbm.at[sid, 0], col_v.at[0], sem_ca)
    pltpu.async_copy(col_hbm.at[sid, 1], col_v.at[1], sem_cb)
    pltpu.async_copy(hs_c.at[row_v.at[0]], buf_a, sem_ga)
    pltpu.async_copy(hs_c.at[row_v.at[1]], buf_b, sem_gb)

    @pl.loop(0, N_CHUNKS, step=2)
    def _(j):
        ja = jnp.minimum(j + 2, N_CHUNKS - 1)
        jb = jnp.minimum(j + 3, N_CHUNKS - 1)
        pltpu.make_async_copy(hs_c.at[row_v.at[0]], buf_a, sem_ga).wait()
        pltpu.make_async_copy(col_hbm.at[sid, 0], col_v.at[0], sem_ca).wait()
        pltpu.async_copy(buf_a, acc_sh.at[col_v.at[0]], sem_sa, add=True)
        pltpu.make_async_copy(hs_c.at[row_v.at[0]], buf_b, sem_gb).wait()
        pltpu.make_async_copy(col_hbm.at[sid, 0], col_v.at[1], sem_cb).wait()
        pltpu.async_copy(buf_b, acc_sh.at[col_v.at[1]], sem_sb, add=True)
        pltpu.make_async_copy(buf_a, acc_sh.at[col_v.at[0]], sem_sa).wait()
        pltpu.async_copy(col_hbm.at[sid, ja], col_v.at[0], sem_ca)
        pltpu.async_copy(hs_c.at[row_v.at[ja]], buf_a, sem_ga)
        pltpu.make_async_copy(buf_b, acc_sh.at[col_v.at[1]], sem_sb).wait()
        pltpu.async_copy(col_hbm.at[sid, jb], col_v.at[1], sem_cb)
        pltpu.async_copy(hs_c.at[row_v.at[jb]], buf_b, sem_gb)

    pltpu.make_async_copy(hs_c.at[row_v.at[0]], buf_a, sem_ga).wait()
    pltpu.make_async_copy(hs_c.at[row_v.at[0]], buf_b, sem_gb).wait()
    pltpu.make_async_copy(col_hbm.at[sid, 0], col_v.at[0], sem_ca).wait()
    pltpu.make_async_copy(col_hbm.at[sid, 0], col_v.at[1], sem_cb).wait()

    plsc.subcore_barrier()
    pltpu.sync_copy(
        acc_sh.at[pl.ds(sid * STRIPE, STRIPE)],
        out_hbm.at[cid, pl.ds(sid * STRIPE, STRIPE)],
    )


_BLK = 2000


def _mm_body(x_ref, wphi_ref, h_ref):
    h_ref[...] = jnp.dot(x_ref[...], wphi_ref[...],
                         preferred_element_type=jnp.float32)


def _mm_call(x, wphi):
    return pl.pallas_call(
        _mm_body,
        grid=(N_NODES // _BLK,),
        in_specs=[
            pl.BlockSpec((_BLK, C), lambda i: (i, 0)),
            pl.BlockSpec((C, C), lambda i: (0, 0)),
        ],
        out_specs=pl.BlockSpec((_BLK, C), lambda i: (i, 0)),
        out_shape=jax.ShapeDtypeStruct((N_NODES, C), jnp.float32),
    )(x, wphi)


def _hs_body(h_ref, deg_ref, hs_ref):
    d = deg_ref[:, 0:1] + deg_ref[:, 1:2] + 1.0
    dinv = lax.rsqrt(d)
    h = h_ref[...]
    hs_ref[0] = dinv * h[:, :HALF]
    hs_ref[1] = dinv * h[:, HALF:]


def _hs_call(h, deg2):
    return pl.pallas_call(
        _hs_body,
        grid=(N_NODES // _BLK,),
        in_specs=[
            pl.BlockSpec((_BLK, C), lambda i: (i, 0)),
            pl.BlockSpec((_BLK, NC), lambda i: (i, 0)),
        ],
        out_specs=pl.BlockSpec((NC, _BLK, HALF), lambda i: (0, i, 0)),
        out_shape=jax.ShapeDtypeStruct((NC, N_NODES, HALF), jnp.float32),
    )(h, deg2)


def _fin_body(x_ref, at_ref, b_ref, acc_ref, deg_ref, o_ref):
    xb = x_ref[...]
    d = deg_ref[:, 0:1] + deg_ref[:, 1:2] + 1.0
    dinv = lax.rsqrt(d)
    z = jnp.dot(xb, at_ref[...], preferred_element_type=jnp.float32) + b_ref[...]
    accf = jnp.concatenate([acc_ref[0], acc_ref[1]], axis=-1)
    o_ref[...] = xb + EPS * jnp.tanh(z + dinv * accf)


def _fin_call(x, w, b2, acc, deg2):
    return pl.pallas_call(
        _fin_body,
        grid=(N_NODES // _BLK,),
        in_specs=[
            pl.BlockSpec((_BLK, C), lambda i: (i, 0)),
            pl.BlockSpec((C, C), lambda i: (0, 0)),
            pl.BlockSpec((1, C), lambda i: (0, 0)),
            pl.BlockSpec((NC, _BLK, HALF), lambda i: (0, i, 0)),
            pl.BlockSpec((_BLK, NC), lambda i: (i, 0)),
        ],
        out_specs=pl.BlockSpec((_BLK, C), lambda i: (i, 0)),
        out_shape=jax.ShapeDtypeStruct((N_NODES, C), jnp.float32),
    )(x, w, b2, acc, deg2)


def kernel(x, edge_index, W, b, W_phi):
    e3 = edge_index.astype(jnp.int32).reshape(2, NS, N_CHUNKS, CHUNK)
    a_t = W.T - W - GAMMA * jnp.eye(C, dtype=W.dtype)

    deg2 = _deg_kernel(e3).T
    h = _mm_call(x, W_phi)
    hs2 = _hs_call(h, deg2)
    acc = _agg_kernel(hs2, e3)
    return _fin_call(x, a_t, b.reshape(1, C), acc, deg2)
